# baseline (speedup 1.0000x reference)
import jax
import jax.numpy as jnp
from jax.experimental import pallas as pl
from jax.experimental.pallas import tpu as pltpu

INPUT_DIM = 768
HIDDEN_DIM = 8192
TOPK = 64
BLOCK_ROWS = 64


def _fused_kernel(x_ref, w_ref, b_ref, dbias_ref, sparse_ref, recon_ref, part_ref,
                  bits2_ref, x2_ref):
    r = pl.program_id(0)
    nb = pl.num_programs(0) - 1
    w = w_ref[...]

    @pl.when(r < nb)
    def _encode():
        x = x_ref[...]
        h = jax.lax.dot_general(
            x, w, (((1,), (1,)), ((), ())),
            preferred_element_type=jnp.float32,
        ) + b_ref[...]
        bits2_ref[r % 2] = jax.lax.bitcast_convert_type(h, jnp.uint32)
        x2_ref[r % 2] = x

    @pl.when(r > 0)
    def _rest():
        slot = (r - 1) % 2
        hb_ref = bits2_ref.at[slot]
        x = x2_ref[slot]
        ba = hb_ref[:, : HIDDEN_DIM // 2]
        bb = hb_ref[:, HIDDEN_DIM // 2:]

        def pack_store(v):
            sparse_ref[:, : HIDDEN_DIM // 2] = jax.lax.bitcast_convert_type(
                v, jnp.float32)

        def pack_load():
            return jax.lax.bitcast_convert_type(
                sparse_ref[:, : HIDDEN_DIM // 2], jnp.uint32)

        pack_store((ba & jnp.uint32(0x7FFF0000))
                   | ((bb >> 16) & jnp.uint32(0x7FFF))
                   | jnp.uint32(0x80008000))

        def count_pair(c):
            diff = pack_load() - ((c << 16) | c)
            contrib = (diff >> 15) & jnp.uint32(0x00010001)
            cnt = jnp.sum(contrib.astype(jnp.int32), axis=1, keepdims=True)
            cnt = cnt.astype(jnp.uint32)
            return (cnt >> 16) + (cnt & jnp.uint32(0xFFFF))

        def body1(i, p):
            c = p | (jnp.uint32(0x4000) >> i.astype(jnp.uint32))
            return jnp.where(count_pair(c) >= TOPK, c, p)

        p = jax.lax.fori_loop(
            0, 15, body1, jnp.zeros((BLOCK_ROWS, 1), jnp.uint32))

        c_hi = count_pair(p + 1)

        fa = jnp.where(((ba >> 16) & jnp.uint32(0x7FFF)) == p,
                       (ba << 15) & jnp.uint32(0x7FFF0000), jnp.uint32(0))
        fb = jnp.where(((bb >> 16) & jnp.uint32(0x7FFF)) == p,
                       (bb >> 1) & jnp.uint32(0x7FFF), jnp.uint32(0))
        pack_store(fa | fb | jnp.uint32(0x80008000))

        def body2(i, t):
            c = t | (jnp.uint32(0x4000) >> i.astype(jnp.uint32))
            return jnp.where(c_hi + count_pair(c) >= TOPK, c, t)

        t = jax.lax.fori_loop(
            0, 15, body2, jnp.zeros((BLOCK_ROWS, 1), jnp.uint32))

        thr0 = (p << 16) | (t << 1)
        hb = hb_ref[...]
        habs = hb & jnp.uint32(0x7FFFFFFF)
        cnt1 = jnp.sum((habs >= (thr0 | 1)).astype(jnp.int32), axis=1,
                       keepdims=True).astype(jnp.uint32)
        thr = jnp.where(cnt1 >= TOPK, thr0 | 1, thr0)

        sparse_ref[...] = jax.lax.bitcast_convert_type(
            jnp.where(habs >= thr, hb, jnp.uint32(0)), jnp.float32)
        s = sparse_ref[...]

        rec = jax.lax.dot_general(
            s, w, (((1,), (0,)), ((), ())),
            preferred_element_type=jnp.float32,
        ) + dbias_ref[...]
        recon_ref[...] = rec

        d = rec - x
        lane = jax.lax.broadcasted_iota(jnp.int32, (1, 1, 128), 2)
        part = (
            jnp.where(lane == 0, jnp.sum(d * d), 0.0)
            + jnp.where(lane == 1, jnp.sum((s != 0.0).astype(jnp.float32), ), 0.0)
            + jnp.where(lane == 2, jnp.sum(jnp.abs(s)), 0.0)
            + jnp.where(lane == 3, jnp.sum(s), 0.0)
            + jnp.where(lane == 4, jnp.max(s), 0.0)
        )
        part_ref[...] = part


@jax.jit
def kernel(x, W_enc, b_enc, dec_bias):
    B = x.shape[0]
    nb = B // BLOCK_ROWS
    sparse, recon, part = pl.pallas_call(
        _fused_kernel,
        grid=(nb + 1,),
        in_specs=[
            pl.BlockSpec((BLOCK_ROWS, INPUT_DIM),
                         lambda r: (jnp.minimum(r, nb - 1), 0)),
            pl.BlockSpec((HIDDEN_DIM, INPUT_DIM), lambda r: (0, 0)),
            pl.BlockSpec((1, HIDDEN_DIM), lambda r: (0, 0)),
            pl.BlockSpec((1, INPUT_DIM), lambda r: (0, 0)),
        ],
        out_specs=[
            pl.BlockSpec((BLOCK_ROWS, HIDDEN_DIM),
                         lambda r: (jnp.maximum(r - 1, 0), 0)),
            pl.BlockSpec((BLOCK_ROWS, INPUT_DIM),
                         lambda r: (jnp.maximum(r - 1, 0), 0)),
            pl.BlockSpec((1, 1, 128),
                         lambda r: (jnp.maximum(r - 1, 0), 0, 0)),
        ],
        out_shape=[
            jax.ShapeDtypeStruct((B, HIDDEN_DIM), jnp.float32),
            jax.ShapeDtypeStruct((B, INPUT_DIM), jnp.float32),
            jax.ShapeDtypeStruct((nb, 1, 128), jnp.float32),
        ],
        scratch_shapes=[
            pltpu.VMEM((2, BLOCK_ROWS, HIDDEN_DIM), jnp.uint32),
            pltpu.VMEM((2, BLOCK_ROWS, INPUT_DIM), jnp.float32),
        ],
        compiler_params=pltpu.CompilerParams(
            dimension_semantics=("arbitrary",),
            vmem_limit_bytes=64 * 1024 * 1024,
        ),
    )(x, W_enc, b_enc.reshape(1, HIDDEN_DIM), dec_bias.reshape(1, INPUT_DIM))

    recon_loss = part[:, 0, 0].sum() / (B * INPUT_DIM)
    num_active = part[:, 0, 1].sum() / B
    sparsity_ratio = num_active / HIDDEN_DIM
    l1_loss = part[:, 0, 2].sum() / (B * HIDDEN_DIM)
    mean_activation = part[:, 0, 3].sum() / (B * HIDDEN_DIM)
    max_activation = part[:, 0, 4].max()
    return (recon, sparse, recon_loss, l1_loss, num_active, sparsity_ratio,
            mean_activation, max_activation)


# speedup vs baseline: 1.5855x; 1.5855x over previous
import functools

import jax
import jax.numpy as jnp
from jax.experimental import pallas as pl
from jax.experimental.pallas import tpu as pltpu

INPUT_DIM = 768
HIDDEN_DIM = 8192
TOPK = 64
BLOCK_ROWS = 128


def _fused_kernel(x_ref, w_ref, b_ref, dbias_ref, sparse_ref, recon_ref, part_ref,
                  bits_ref, pack_ref):
    x = x_ref[...]
    w = w_ref[...]
    h = jax.lax.dot_general(
        x, w, (((1,), (1,)), ((), ())),
        preferred_element_type=jnp.float32,
    ) + b_ref[...]
    br = x.shape[0]

    bits_ref[...] = jax.lax.bitcast_convert_type(h, jnp.uint32)
    ba = bits_ref[:, : HIDDEN_DIM // 2]
    bb = bits_ref[:, HIDDEN_DIM // 2:]

    pack_ref[...] = ((ba & jnp.uint32(0x7FFF0000))
                     | ((bb >> 16) & jnp.uint32(0x7FFF))
                     | jnp.uint32(0x80008000))

    def count_pair(c):
        diff = pack_ref[...] - ((c << 16) | c)
        contrib = (diff >> 15) & jnp.uint32(0x00010001)
        cnt = jnp.sum(contrib.astype(jnp.int32), axis=1, keepdims=True)
        cnt = cnt.astype(jnp.uint32)
        return (cnt >> 16) + (cnt & jnp.uint32(0xFFFF))

    def body1(i, p):
        c = p | (jnp.uint32(0x4000) >> i.astype(jnp.uint32))
        return jnp.where(count_pair(c) >= TOPK, c, p)

    p = jax.lax.fori_loop(
        0, 15, body1, jnp.zeros((br, 1), jnp.uint32), unroll=3)

    c_hi = count_pair(p + 1)

    fa = jnp.where(((ba >> 16) & jnp.uint32(0x7FFF)) == p,
                   (ba << 15) & jnp.uint32(0x7FFF0000), jnp.uint32(0))
    fb = jnp.where(((bb >> 16) & jnp.uint32(0x7FFF)) == p,
                   (bb >> 1) & jnp.uint32(0x7FFF), jnp.uint32(0))
    pack_ref[...] = fa | fb | jnp.uint32(0x80008000)

    def body2(i, t):
        c = t | (jnp.uint32(0x4000) >> i.astype(jnp.uint32))
        return jnp.where(c_hi + count_pair(c) >= TOPK, c, t)

    t = jax.lax.fori_loop(
        0, 15, body2, jnp.zeros((br, 1), jnp.uint32), unroll=3)

    thr0 = (p << 16) | (t << 1)
    hb = bits_ref[...]
    habs = hb & jnp.uint32(0x7FFFFFFF)
    cnt1 = jnp.sum((habs >= (thr0 | 1)).astype(jnp.int32), axis=1,
                   keepdims=True).astype(jnp.uint32)
    thr = jnp.where(cnt1 >= TOPK, thr0 | 1, thr0)

    sparse_ref[...] = jax.lax.bitcast_convert_type(
        jnp.where(habs >= thr, hb, jnp.uint32(0)), jnp.float32)
    s = sparse_ref[...]

    r = jax.lax.dot_general(
        s, w, (((1,), (0,)), ((), ())),
        preferred_element_type=jnp.float32,
    ) + dbias_ref[...]
    recon_ref[...] = r

    d = r - x
    lane = jax.lax.broadcasted_iota(jnp.int32, (1, 1, 128), 2)
    part = (
        jnp.where(lane == 0, jnp.sum(d * d), 0.0)
        + jnp.where(lane == 1, jnp.sum((s != 0.0).astype(jnp.float32)), 0.0)
        + jnp.where(lane == 2, jnp.sum(jnp.abs(s)), 0.0)
        + jnp.where(lane == 3, jnp.sum(s), 0.0)
        + jnp.where(lane == 4, jnp.max(s), 0.0)
    )
    part_ref[...] = part


@jax.jit
def kernel(x, W_enc, b_enc, dec_bias):
    B = x.shape[0]
    nb = B // BLOCK_ROWS
    sparse, recon, part = pl.pallas_call(
        _fused_kernel,
        grid=(nb,),
        in_specs=[
            pl.BlockSpec((BLOCK_ROWS, INPUT_DIM), lambda r: (r, 0)),
            pl.BlockSpec((HIDDEN_DIM, INPUT_DIM), lambda r: (0, 0)),
            pl.BlockSpec((1, HIDDEN_DIM), lambda r: (0, 0)),
            pl.BlockSpec((1, INPUT_DIM), lambda r: (0, 0)),
        ],
        out_specs=[
            pl.BlockSpec((BLOCK_ROWS, HIDDEN_DIM), lambda r: (r, 0)),
            pl.BlockSpec((BLOCK_ROWS, INPUT_DIM), lambda r: (r, 0)),
            pl.BlockSpec((1, 1, 128), lambda r: (r, 0, 0)),
        ],
        out_shape=[
            jax.ShapeDtypeStruct((B, HIDDEN_DIM), jnp.float32),
            jax.ShapeDtypeStruct((B, INPUT_DIM), jnp.float32),
            jax.ShapeDtypeStruct((nb, 1, 128), jnp.float32),
        ],
        scratch_shapes=[
            pltpu.VMEM((BLOCK_ROWS, HIDDEN_DIM), jnp.uint32),
            pltpu.VMEM((BLOCK_ROWS, HIDDEN_DIM // 2), jnp.uint32),
        ],
        compiler_params=pltpu.CompilerParams(
            dimension_semantics=("parallel",),
            vmem_limit_bytes=64 * 1024 * 1024,
        ),
    )(x, W_enc, b_enc.reshape(1, HIDDEN_DIM), dec_bias.reshape(1, INPUT_DIM))

    recon_loss = part[:, 0, 0].sum() / (B * INPUT_DIM)
    num_active = part[:, 0, 1].sum() / B
    sparsity_ratio = num_active / HIDDEN_DIM
    l1_loss = part[:, 0, 2].sum() / (B * HIDDEN_DIM)
    mean_activation = part[:, 0, 3].sum() / (B * HIDDEN_DIM)
    max_activation = part[:, 0, 4].max()
    return (recon, sparse, recon_loss, l1_loss, num_active, sparsity_ratio,
            mean_activation, max_activation)


# speedup vs baseline: 1.5890x; 1.0022x over previous
import functools

import jax
import jax.numpy as jnp
from jax.experimental import pallas as pl
from jax.experimental.pallas import tpu as pltpu

INPUT_DIM = 768
HIDDEN_DIM = 8192
TOPK = 64
BLOCK_ROWS = 128


def _fused_kernel(x_ref, w_ref, b_ref, dbias_ref, sparse_ref, recon_ref, part_ref,
                  bits_ref, pack_ref):
    x = x_ref[...]
    w = w_ref[...]
    h = jax.lax.dot_general(
        x, w, (((1,), (1,)), ((), ())),
        preferred_element_type=jnp.float32,
    ) + b_ref[...]
    br = x.shape[0]

    bits_ref[...] = jax.lax.bitcast_convert_type(h, jnp.uint32)
    ba = bits_ref[:, : HIDDEN_DIM // 2]
    bb = bits_ref[:, HIDDEN_DIM // 2:]

    pack_ref[...] = ((ba & jnp.uint32(0x7FFF0000))
                     | ((bb >> 16) & jnp.uint32(0x7FFF))
                     | jnp.uint32(0x80008000))

    def count_pair(c):
        diff = pack_ref[...] - ((c << 16) | c)
        contrib = (diff >> 15) & jnp.uint32(0x00010001)
        cnt = jnp.sum(contrib.astype(jnp.int32), axis=1, keepdims=True)
        cnt = cnt.astype(jnp.uint32)
        return (cnt >> 16) + (cnt & jnp.uint32(0xFFFF))

    def body1(i, p):
        c = p | (jnp.uint32(0x4000) >> i.astype(jnp.uint32))
        return jnp.where(count_pair(c) >= TOPK, c, p)

    p = jax.lax.fori_loop(
        0, 15, body1, jnp.zeros((br, 1), jnp.uint32), unroll=5)

    c_hi = count_pair(p + 1)

    fa = jnp.where(((ba >> 16) & jnp.uint32(0x7FFF)) == p,
                   (ba << 15) & jnp.uint32(0x7FFF0000), jnp.uint32(0))
    fb = jnp.where(((bb >> 16) & jnp.uint32(0x7FFF)) == p,
                   (bb >> 1) & jnp.uint32(0x7FFF), jnp.uint32(0))
    pack_ref[...] = fa | fb | jnp.uint32(0x80008000)

    def body2(i, t):
        c = t | (jnp.uint32(0x4000) >> i.astype(jnp.uint32))
        return jnp.where(c_hi + count_pair(c) >= TOPK, c, t)

    t = jax.lax.fori_loop(
        0, 15, body2, jnp.zeros((br, 1), jnp.uint32), unroll=5)

    thr0 = (p << 16) | (t << 1)
    hb = bits_ref[...]
    habs = hb & jnp.uint32(0x7FFFFFFF)
    cnt1 = jnp.sum((habs >= (thr0 | 1)).astype(jnp.int32), axis=1,
                   keepdims=True).astype(jnp.uint32)
    thr = jnp.where(cnt1 >= TOPK, thr0 | 1, thr0)

    sparse_ref[...] = jax.lax.bitcast_convert_type(
        jnp.where(habs >= thr, hb, jnp.uint32(0)), jnp.float32)
    s = sparse_ref[...]

    r = jax.lax.dot_general(
        s, w, (((1,), (0,)), ((), ())),
        preferred_element_type=jnp.float32,
    ) + dbias_ref[...]
    recon_ref[...] = r

    d = r - x
    lane = jax.lax.broadcasted_iota(jnp.int32, (1, 1, 128), 2)
    part = (
        jnp.where(lane == 0, jnp.sum(d * d), 0.0)
        + jnp.where(lane == 1, jnp.sum((s != 0.0).astype(jnp.float32)), 0.0)
        + jnp.where(lane == 2, jnp.sum(jnp.abs(s)), 0.0)
        + jnp.where(lane == 3, jnp.sum(s), 0.0)
        + jnp.where(lane == 4, jnp.max(s), 0.0)
    )
    part_ref[...] = part


@jax.jit
def kernel(x, W_enc, b_enc, dec_bias):
    B = x.shape[0]
    nb = B // BLOCK_ROWS
    sparse, recon, part = pl.pallas_call(
        _fused_kernel,
        grid=(nb,),
        in_specs=[
            pl.BlockSpec((BLOCK_ROWS, INPUT_DIM), lambda r: (r, 0)),
            pl.BlockSpec((HIDDEN_DIM, INPUT_DIM), lambda r: (0, 0)),
            pl.BlockSpec((1, HIDDEN_DIM), lambda r: (0, 0)),
            pl.BlockSpec((1, INPUT_DIM), lambda r: (0, 0)),
        ],
        out_specs=[
            pl.BlockSpec((BLOCK_ROWS, HIDDEN_DIM), lambda r: (r, 0)),
            pl.BlockSpec((BLOCK_ROWS, INPUT_DIM), lambda r: (r, 0)),
            pl.BlockSpec((1, 1, 128), lambda r: (r, 0, 0)),
        ],
        out_shape=[
            jax.ShapeDtypeStruct((B, HIDDEN_DIM), jnp.float32),
            jax.ShapeDtypeStruct((B, INPUT_DIM), jnp.float32),
            jax.ShapeDtypeStruct((nb, 1, 128), jnp.float32),
        ],
        scratch_shapes=[
            pltpu.VMEM((BLOCK_ROWS, HIDDEN_DIM), jnp.uint32),
            pltpu.VMEM((BLOCK_ROWS, HIDDEN_DIM // 2), jnp.uint32),
        ],
        compiler_params=pltpu.CompilerParams(
            dimension_semantics=("parallel",),
            vmem_limit_bytes=64 * 1024 * 1024,
        ),
    )(x, W_enc, b_enc.reshape(1, HIDDEN_DIM), dec_bias.reshape(1, INPUT_DIM))

    recon_loss = part[:, 0, 0].sum() / (B * INPUT_DIM)
    num_active = part[:, 0, 1].sum() / B
    sparsity_ratio = num_active / HIDDEN_DIM
    l1_loss = part[:, 0, 2].sum() / (B * HIDDEN_DIM)
    mean_activation = part[:, 0, 3].sum() / (B * HIDDEN_DIM)
    max_activation = part[:, 0, 4].max()
    return (recon, sparse, recon_loss, l1_loss, num_active, sparsity_ratio,
            mean_activation, max_activation)


# speedup vs baseline: 1.5893x; 1.0001x over previous
import jax
import jax.numpy as jnp
from jax.experimental import pallas as pl
from jax.experimental.pallas import tpu as pltpu

INPUT_DIM = 768
HIDDEN_DIM = 8192
TOPK = 64
BLOCK_ROWS = 128


def _fused_kernel(x_ref, w_ref, b_ref, dbias_ref, sparse_ref, recon_ref, part_ref,
                  bits_ref, pack_ref):
    x = x_ref[...]
    w = w_ref[...]
    h = jax.lax.dot_general(
        x, w, (((1,), (1,)), ((), ())),
        preferred_element_type=jnp.float32,
    ) + b_ref[...]
    br = x.shape[0]

    bits_ref[...] = jax.lax.bitcast_convert_type(h, jnp.uint32)
    ba = bits_ref[:, : HIDDEN_DIM // 2]
    bb = bits_ref[:, HIDDEN_DIM // 2:]

    pack_ref[...] = ((ba & jnp.uint32(0x7FFF0000))
                     | ((bb >> 16) & jnp.uint32(0x7FFF))
                     | jnp.uint32(0x80008000))

    def count_pair(c):
        diff = pack_ref[...] - ((c << 16) | c)
        contrib = (diff >> 15) & jnp.uint32(0x00010001)
        cnt = jnp.sum(contrib.astype(jnp.int32), axis=1, keepdims=True)
        cnt = cnt.astype(jnp.uint32)
        return (cnt >> 16) + (cnt & jnp.uint32(0xFFFF))

    def body1(i, p):
        c = p | (jnp.uint32(0x4000) >> i.astype(jnp.uint32))
        return jnp.where(count_pair(c) >= TOPK, c, p)

    p = jax.lax.fori_loop(
        0, 15, body1, jnp.zeros((br, 1), jnp.uint32), unroll=5)

    c_hi = count_pair(p + 1)

    fa = jnp.where(((ba >> 16) & jnp.uint32(0x7FFF)) == p,
                   (ba << 15) & jnp.uint32(0x7FFF0000), jnp.uint32(0))
    fb = jnp.where(((bb >> 16) & jnp.uint32(0x7FFF)) == p,
                   (bb >> 1) & jnp.uint32(0x7FFF), jnp.uint32(0))
    pack_ref[...] = fa | fb | jnp.uint32(0x80008000)

    def body2(i, t):
        c = t | (jnp.uint32(0x4000) >> i.astype(jnp.uint32))
        return jnp.where(c_hi + count_pair(c) >= TOPK, c, t)

    t = jax.lax.fori_loop(
        0, 15, body2, jnp.zeros((br, 1), jnp.uint32), unroll=5)

    thr0 = (p << 16) | (t << 1)
    hb = bits_ref[...]
    habs = hb & jnp.uint32(0x7FFFFFFF)
    cnt1 = jnp.sum((habs >= (thr0 | 1)).astype(jnp.int32), axis=1,
                   keepdims=True).astype(jnp.uint32)
    thr = jnp.where(cnt1 >= TOPK, thr0 | 1, thr0)

    sparse_ref[...] = jax.lax.bitcast_convert_type(
        jnp.where(habs >= thr, hb, jnp.uint32(0)), jnp.float32)
    s = sparse_ref[...]

    r = jax.lax.dot_general(
        s, w, (((1,), (0,)), ((), ())),
        preferred_element_type=jnp.float32,
    ) + dbias_ref[...]
    recon_ref[...] = r

    d = r - x
    lane = jax.lax.broadcasted_iota(jnp.int32, (1, 1, 128), 2)
    part = (
        jnp.where(lane == 0, jnp.sum(d * d), 0.0)
        + jnp.where(lane == 1, jnp.sum((s != 0.0).astype(jnp.float32)), 0.0)
        + jnp.where(lane == 2, jnp.sum(jnp.abs(s)), 0.0)
        + jnp.where(lane == 3, jnp.sum(s), 0.0)
        + jnp.where(lane == 4, jnp.max(s), 0.0)
    )
    part_ref[...] = part


@jax.jit
def kernel(x, W_enc, b_enc, dec_bias):
    B = x.shape[0]
    nb = B // BLOCK_ROWS
    sparse, recon, part = pl.pallas_call(
        _fused_kernel,
        grid=(nb,),
        in_specs=[
            pl.BlockSpec((BLOCK_ROWS, INPUT_DIM), lambda r: (r, 0)),
            pl.BlockSpec((HIDDEN_DIM, INPUT_DIM), lambda r: (0, 0)),
            pl.BlockSpec((1, HIDDEN_DIM), lambda r: (0, 0)),
            pl.BlockSpec((1, INPUT_DIM), lambda r: (0, 0)),
        ],
        out_specs=[
            pl.BlockSpec((BLOCK_ROWS, HIDDEN_DIM), lambda r: (r, 0)),
            pl.BlockSpec((BLOCK_ROWS, INPUT_DIM), lambda r: (r, 0)),
            pl.BlockSpec((1, 1, 128), lambda r: (r, 0, 0)),
        ],
        out_shape=[
            jax.ShapeDtypeStruct((B, HIDDEN_DIM), jnp.float32),
            jax.ShapeDtypeStruct((B, INPUT_DIM), jnp.float32),
            jax.ShapeDtypeStruct((nb, 1, 128), jnp.float32),
        ],
        scratch_shapes=[
            pltpu.VMEM((BLOCK_ROWS, HIDDEN_DIM), jnp.uint32),
            pltpu.VMEM((BLOCK_ROWS, HIDDEN_DIM // 2), jnp.uint32),
        ],
        compiler_params=pltpu.CompilerParams(
            dimension_semantics=("parallel",),
            vmem_limit_bytes=64 * 1024 * 1024,
        ),
    )(x, W_enc, b_enc.reshape(1, HIDDEN_DIM), dec_bias.reshape(1, INPUT_DIM))

    recon_loss = part[:, 0, 0].sum() / (B * INPUT_DIM)
    num_active = part[:, 0, 1].sum() / B
    sparsity_ratio = num_active / HIDDEN_DIM
    l1_loss = part[:, 0, 2].sum() / (B * HIDDEN_DIM)
    mean_activation = part[:, 0, 3].sum() / (B * HIDDEN_DIM)
    max_activation = part[:, 0, 4].max()
    return (recon, sparse, recon_loss, l1_loss, num_active, sparsity_ratio,
            mean_activation, max_activation)
